# baseline (speedup 1.0000x reference)
import functools

import jax
import jax.numpy as jnp
from jax import lax
from jax.experimental import pallas as pl
from jax.experimental.pallas import tpu as pltpu
from jax.experimental.pallas import tpu_sc as plsc

B = 16384
MF_D = 16
MLP_D = 10
NC = 2
NS = 16
NW = NC * NS
BSC = 8192
BTC = B - BSC
BPW = BSC // NW
RR = 128


@functools.cache
def _make_sc_gather():
    mesh = plsc.VectorSubcoreMesh(core_axis_name="c", subcore_axis_name="s")

    @functools.partial(
        pl.kernel,
        mesh=mesh,
        compiler_params=pltpu.CompilerParams(use_tc_tiling_on_sc=True),
        out_type=[
            jax.ShapeDtypeStruct((BSC, MF_D), jnp.float32),
            jax.ShapeDtypeStruct((BSC, MF_D), jnp.float32),
            jax.ShapeDtypeStruct((BSC, MLP_D), jnp.float32),
            jax.ShapeDtypeStruct((BSC, MLP_D), jnp.float32),
        ],
        scratch_types=[
            pltpu.VMEM((BPW,), jnp.int32),
            pltpu.VMEM((BPW,), jnp.int32),
            pltpu.VMEM((RR, MF_D), jnp.float32),
            pltpu.VMEM((RR, MF_D), jnp.float32),
            pltpu.VMEM((RR, MLP_D), jnp.float32),
            pltpu.VMEM((RR, MLP_D), jnp.float32),
            pltpu.SemaphoreType.DMA,
        ],
    )
    def _sc_gather(u_hbm, i_hbm, mfu_hbm, mfi_hbm, mlpu_hbm, mlpi_hbm,
                   o_mfu, o_mfi, o_mlpu, o_mlpi,
                   uv, iv, bmfu, bmfi, bmlpu, bmlpi, sem):
        wid = lax.axis_index("s") * NC + lax.axis_index("c")
        base = wid * BPW
        pltpu.sync_copy(u_hbm.at[pl.ds(base, BPW)], uv)
        pltpu.sync_copy(i_hbm.at[pl.ds(base, BPW)], iv)

        def round_(t, _):
            r0 = t * RR

            def body(g, _):
                gr = r0 + g * 16
                uvec = uv[pl.ds(gr, 16)]
                ivec = iv[pl.ds(gr, 16)]
                for j in range(16):
                    r = g * 16 + j
                    a = uvec[j]
                    b = ivec[j]
                    pltpu.async_copy(mfu_hbm.at[pl.ds(a, 1)],
                                     bmfu.at[pl.ds(r, 1)], sem)
                    pltpu.async_copy(mfi_hbm.at[pl.ds(b, 1)],
                                     bmfi.at[pl.ds(r, 1)], sem)
                    pltpu.async_copy(mlpu_hbm.at[pl.ds(a, 1)],
                                     bmlpu.at[pl.ds(r, 1)], sem)
                    pltpu.async_copy(mlpi_hbm.at[pl.ds(b, 1)],
                                     bmlpi.at[pl.ds(r, 1)], sem)
                return _

            lax.fori_loop(0, RR // 16, body, 0)
            ob = base + r0
            pltpu.make_async_copy(o_mfu.at[pl.ds(ob, RR)], bmfu, sem).wait()
            pltpu.make_async_copy(o_mfi.at[pl.ds(ob, RR)], bmfi, sem).wait()
            pltpu.make_async_copy(o_mlpu.at[pl.ds(ob, RR)], bmlpu, sem).wait()
            pltpu.make_async_copy(o_mlpi.at[pl.ds(ob, RR)], bmlpi, sem).wait()
            pltpu.sync_copy(bmfu, o_mfu.at[pl.ds(ob, RR)])
            pltpu.sync_copy(bmfi, o_mfi.at[pl.ds(ob, RR)])
            pltpu.sync_copy(bmlpu, o_mlpu.at[pl.ds(ob, RR)])
            pltpu.sync_copy(bmlpi, o_mlpi.at[pl.ds(ob, RR)])
            return _

        lax.fori_loop(0, BPW // RR, round_, 0)

    return _sc_gather


def _tc_gather_body(u_smem, i_smem, mfu_hbm, mfi_hbm, mlpu_hbm, mlpi_hbm,
                    o_mfu, o_mfi, o_mlpu, o_mlpi,
                    bmfu, bmfi, bmlpu, bmlpi, sem):
    def body(r, _):
        a = u_smem[r]
        b = i_smem[r]
        pltpu.make_async_copy(mfu_hbm.at[pl.ds(a, 1)],
                              bmfu.at[pl.ds(r, 1)], sem).start()
        pltpu.make_async_copy(mfi_hbm.at[pl.ds(b, 1)],
                              bmfi.at[pl.ds(r, 1)], sem).start()
        pltpu.make_async_copy(mlpu_hbm.at[pl.ds(a, 1)],
                              bmlpu.at[pl.ds(r, 1)], sem).start()
        pltpu.make_async_copy(mlpi_hbm.at[pl.ds(b, 1)],
                              bmlpi.at[pl.ds(r, 1)], sem).start()
        return _

    lax.fori_loop(0, BTC, body, 0, unroll=8)
    pltpu.make_async_copy(mfu_hbm.at[pl.ds(0, BTC)], bmfu, sem).wait()
    pltpu.make_async_copy(mfi_hbm.at[pl.ds(0, BTC)], bmfi, sem).wait()
    pltpu.make_async_copy(mlpu_hbm.at[pl.ds(0, BTC)], bmlpu, sem).wait()
    pltpu.make_async_copy(mlpi_hbm.at[pl.ds(0, BTC)], bmlpi, sem).wait()
    o_mfu[...] = bmfu[...]
    o_mfi[...] = bmfi[...]
    o_mlpu[...] = bmlpu[...]
    o_mlpi[...] = bmlpi[...]


def _tc_gather(u2, i2, mf_user, mf_item, mlp_user, mlp_item):
    hbm = pl.BlockSpec(memory_space=pltpu.MemorySpace.HBM)
    smem = pl.BlockSpec(memory_space=pltpu.SMEM)
    return pl.pallas_call(
        _tc_gather_body,
        in_specs=[smem, smem, hbm, hbm, hbm, hbm],
        out_specs=[
            pl.BlockSpec((BTC, MF_D), lambda: (0, 0)),
            pl.BlockSpec((BTC, MF_D), lambda: (0, 0)),
            pl.BlockSpec((BTC, MLP_D), lambda: (0, 0)),
            pl.BlockSpec((BTC, MLP_D), lambda: (0, 0)),
        ],
        out_shape=[
            jax.ShapeDtypeStruct((BTC, MF_D), jnp.float32),
            jax.ShapeDtypeStruct((BTC, MF_D), jnp.float32),
            jax.ShapeDtypeStruct((BTC, MLP_D), jnp.float32),
            jax.ShapeDtypeStruct((BTC, MLP_D), jnp.float32),
        ],
        scratch_shapes=[
            pltpu.VMEM((BTC, MF_D), jnp.float32),
            pltpu.VMEM((BTC, MF_D), jnp.float32),
            pltpu.VMEM((BTC, MLP_D), jnp.float32),
            pltpu.VMEM((BTC, MLP_D), jnp.float32),
            pltpu.SemaphoreType.DMA,
        ],
    )(u2, i2, mf_user, mf_item, mlp_user, mlp_item)


BM = 2048


def _tc_mlp_body(mfu, mfi, mlpu, mlpi, W1a, W1b, b1, W2, b2, W3, b3,
                 Wl, bl, w2la, w2lb, b2l, out):
    f32 = jnp.float32
    x = (jnp.dot(mlpu[...], W1a[...], preferred_element_type=f32)
         + jnp.dot(mlpi[...], W1b[...], preferred_element_type=f32)
         + b1[...])
    x = jnp.maximum(x, 0.0)
    x = jnp.dot(x, W2[...], preferred_element_type=f32) + b2[...]
    x = jnp.maximum(x, 0.0)
    x = jnp.dot(x, W3[...], preferred_element_type=f32) + b3[...]
    x = jnp.maximum(x, 0.0)
    mlp_vec = jnp.dot(x, Wl[...], preferred_element_type=f32) + bl[...]
    mf_vec = mfu[...] * mfi[...]
    logit = (jnp.dot(mf_vec, w2la[...], preferred_element_type=f32)
             + jnp.dot(mlp_vec, w2lb[...], preferred_element_type=f32)
             + b2l[...])
    out[...] = jax.nn.sigmoid(logit)


def _tc_mlp(mfu, mfi, mlpu, mlpi, W1a, W1b, b1, W2, b2, W3, b3,
            Wl, bl, w2la, w2lb, b2l):
    def row_block(d):
        return pl.BlockSpec((BM, d), lambda m: (m, 0))

    def full(a):
        return pl.BlockSpec(a.shape, lambda m: (0,) * a.ndim)

    return pl.pallas_call(
        _tc_mlp_body,
        grid=(B // BM,),
        in_specs=[
            row_block(MF_D), row_block(MF_D), row_block(MLP_D),
            row_block(MLP_D),
            full(W1a), full(W1b), full(b1), full(W2), full(b2),
            full(W3), full(b3), full(Wl), full(bl),
            full(w2la), full(w2lb), full(b2l),
        ],
        out_specs=pl.BlockSpec((BM, 1), lambda m: (m, 0)),
        out_shape=jax.ShapeDtypeStruct((B, 1), jnp.float32),
    )(mfu, mfi, mlpu, mlpi, W1a, W1b, b1, W2, b2, W3, b3,
      Wl, bl, w2la, w2lb, b2l)


def kernel(inputs, mf_user, mf_item, mlp_user, mlp_item,
           W1, b1, W2, b2, W3, b3, Wl, bl, W2l, b2l):
    u = inputs[:, 0]
    i = inputs[:, 1]
    mfu1, mfi1, mlpu1, mlpi1 = _make_sc_gather()(
        u[:BSC], i[:BSC], mf_user, mf_item, mlp_user, mlp_item)
    mfu2, mfi2, mlpu2, mlpi2 = _tc_gather(
        u[BSC:], i[BSC:], mf_user, mf_item, mlp_user, mlp_item)
    mfu = jnp.concatenate([mfu1, mfu2], axis=0)
    mfi = jnp.concatenate([mfi1, mfi2], axis=0)
    mlpu = jnp.concatenate([mlpu1, mlpu2], axis=0)
    mlpi = jnp.concatenate([mlpi1, mlpi2], axis=0)
    return _tc_mlp(
        mfu, mfi, mlpu, mlpi,
        W1[:MLP_D], W1[MLP_D:], b1.reshape(1, -1),
        W2, b2.reshape(1, -1), W3, b3.reshape(1, -1),
        Wl, bl.reshape(1, -1),
        W2l[:MF_D], W2l[MF_D:], b2l.reshape(1, 1),
    )


# speedup vs baseline: 1.0295x; 1.0295x over previous
import functools

import jax
import jax.numpy as jnp
from jax import lax
from jax.experimental import pallas as pl
from jax.experimental.pallas import tpu as pltpu
from jax.experimental.pallas import tpu_sc as plsc

B = 16384
MF_D = 16
MLP_D = 10
NC = 2
NS = 16
NW = NC * NS
BPW = B // NW
RR = 128


@functools.cache
def _make_sc_gather():
    mesh = plsc.VectorSubcoreMesh(core_axis_name="c", subcore_axis_name="s")

    @functools.partial(
        pl.kernel,
        mesh=mesh,
        compiler_params=pltpu.CompilerParams(use_tc_tiling_on_sc=True),
        out_type=[
            jax.ShapeDtypeStruct((B, MF_D), jnp.float32),
            jax.ShapeDtypeStruct((B, MF_D), jnp.float32),
            jax.ShapeDtypeStruct((B, MLP_D), jnp.float32),
            jax.ShapeDtypeStruct((B, MLP_D), jnp.float32),
        ],
        scratch_types=[
            pltpu.VMEM((BPW,), jnp.int32),
            pltpu.VMEM((BPW,), jnp.int32),
            pltpu.VMEM((RR, MF_D), jnp.float32),
            pltpu.VMEM((RR, MF_D), jnp.float32),
            pltpu.VMEM((RR, MLP_D), jnp.float32),
            pltpu.VMEM((RR, MLP_D), jnp.float32),
        ] + [pltpu.SemaphoreType.DMA] * 16,
    )
    def _sc_gather(u_hbm, i_hbm, mfu_hbm, mfi_hbm, mlpu_hbm, mlpi_hbm,
                   o_mfu, o_mfi, o_mlpu, o_mlpi,
                   uv, iv, bmfu, bmfi, bmlpu, bmlpi, *sems):
        wid = lax.axis_index("s") * NC + lax.axis_index("c")
        base = wid * BPW
        pltpu.sync_copy(u_hbm.at[pl.ds(base, BPW)], uv)
        pltpu.sync_copy(i_hbm.at[pl.ds(base, BPW)], iv)

        def round_(t, _):
            r0 = t * RR

            def body(g, _):
                gr = r0 + g * 16
                uvec = uv[pl.ds(gr, 16)]
                ivec = iv[pl.ds(gr, 16)]
                for j in range(16):
                    r = g * 16 + j
                    q = j % 4
                    a = uvec[j]
                    b = ivec[j]
                    pltpu.async_copy(mfu_hbm.at[pl.ds(a, 1)],
                                     bmfu.at[pl.ds(r, 1)], sems[q])
                    pltpu.async_copy(mfi_hbm.at[pl.ds(b, 1)],
                                     bmfi.at[pl.ds(r, 1)], sems[4 + q])
                    pltpu.async_copy(mlpu_hbm.at[pl.ds(a, 1)],
                                     bmlpu.at[pl.ds(r, 1)], sems[8 + q])
                    pltpu.async_copy(mlpi_hbm.at[pl.ds(b, 1)],
                                     bmlpi.at[pl.ds(r, 1)], sems[12 + q])
                return _

            lax.fori_loop(0, RR // 16, body, 0)
            ob = base + r0
            q4 = RR // 4
            for q in range(4):
                pltpu.make_async_copy(o_mfu.at[pl.ds(ob, q4)],
                                      bmfu.at[pl.ds(0, q4)], sems[q]).wait()
                pltpu.make_async_copy(o_mfi.at[pl.ds(ob, q4)],
                                      bmfi.at[pl.ds(0, q4)],
                                      sems[4 + q]).wait()
                pltpu.make_async_copy(o_mlpu.at[pl.ds(ob, q4)],
                                      bmlpu.at[pl.ds(0, q4)],
                                      sems[8 + q]).wait()
                pltpu.make_async_copy(o_mlpi.at[pl.ds(ob, q4)],
                                      bmlpi.at[pl.ds(0, q4)],
                                      sems[12 + q]).wait()
            pltpu.sync_copy(bmfu, o_mfu.at[pl.ds(ob, RR)])
            pltpu.sync_copy(bmfi, o_mfi.at[pl.ds(ob, RR)])
            pltpu.sync_copy(bmlpu, o_mlpu.at[pl.ds(ob, RR)])
            pltpu.sync_copy(bmlpi, o_mlpi.at[pl.ds(ob, RR)])
            return _

        lax.fori_loop(0, BPW // RR, round_, 0)

    return _sc_gather


BM = 2048


def _tc_mlp_body(mfu, mfi, mlpu, mlpi, W1a, W1b, b1, W2, b2, W3, b3,
                 Wl, bl, w2la, w2lb, b2l, out):
    f32 = jnp.float32
    x = (jnp.dot(mlpu[...], W1a[...], preferred_element_type=f32)
         + jnp.dot(mlpi[...], W1b[...], preferred_element_type=f32)
         + b1[...])
    x = jnp.maximum(x, 0.0)
    x = jnp.dot(x, W2[...], preferred_element_type=f32) + b2[...]
    x = jnp.maximum(x, 0.0)
    x = jnp.dot(x, W3[...], preferred_element_type=f32) + b3[...]
    x = jnp.maximum(x, 0.0)
    mlp_vec = jnp.dot(x, Wl[...], preferred_element_type=f32) + bl[...]
    mf_vec = mfu[...] * mfi[...]
    logit = (jnp.dot(mf_vec, w2la[...], preferred_element_type=f32)
             + jnp.dot(mlp_vec, w2lb[...], preferred_element_type=f32)
             + b2l[...])
    out[...] = jax.nn.sigmoid(logit)


def _tc_mlp(mfu, mfi, mlpu, mlpi, W1a, W1b, b1, W2, b2, W3, b3,
            Wl, bl, w2la, w2lb, b2l):
    def row_block(d):
        return pl.BlockSpec((BM, d), lambda m: (m, 0))

    def full(a):
        return pl.BlockSpec(a.shape, lambda m: (0,) * a.ndim)

    return pl.pallas_call(
        _tc_mlp_body,
        grid=(B // BM,),
        in_specs=[
            row_block(MF_D), row_block(MF_D), row_block(MLP_D),
            row_block(MLP_D),
            full(W1a), full(W1b), full(b1), full(W2), full(b2),
            full(W3), full(b3), full(Wl), full(bl),
            full(w2la), full(w2lb), full(b2l),
        ],
        out_specs=pl.BlockSpec((BM, 1), lambda m: (m, 0)),
        out_shape=jax.ShapeDtypeStruct((B, 1), jnp.float32),
    )(mfu, mfi, mlpu, mlpi, W1a, W1b, b1, W2, b2, W3, b3,
      Wl, bl, w2la, w2lb, b2l)


def kernel(inputs, mf_user, mf_item, mlp_user, mlp_item,
           W1, b1, W2, b2, W3, b3, Wl, bl, W2l, b2l):
    u = inputs[:, 0]
    i = inputs[:, 1]
    mfu, mfi, mlpu, mlpi = _make_sc_gather()(
        u, i, mf_user, mf_item, mlp_user, mlp_item)
    return _tc_mlp(
        mfu, mfi, mlpu, mlpi,
        W1[:MLP_D], W1[MLP_D:], b1.reshape(1, -1),
        W2, b2.reshape(1, -1), W3, b3.reshape(1, -1),
        Wl, bl.reshape(1, -1),
        W2l[:MF_D], W2l[MF_D:], b2l.reshape(1, 1),
    )


# speedup vs baseline: 1.0579x; 1.0276x over previous
import functools

import jax
import jax.numpy as jnp
from jax import lax
from jax.experimental import pallas as pl
from jax.experimental.pallas import tpu as pltpu
from jax.experimental.pallas import tpu_sc as plsc

B = 16384
MF_D = 16
MLP_D = 10
NC = 2
NS = 16
NW = NC * NS
BSC = 12288
BTC = B - BSC
BPW = BSC // NW
RR = 128


@functools.cache
def _make_sc_gather():
    mesh = plsc.VectorSubcoreMesh(core_axis_name="c", subcore_axis_name="s")

    @functools.partial(
        pl.kernel,
        mesh=mesh,
        compiler_params=pltpu.CompilerParams(use_tc_tiling_on_sc=True),
        out_type=[
            jax.ShapeDtypeStruct((BSC, MF_D), jnp.float32),
            jax.ShapeDtypeStruct((BSC, MF_D), jnp.float32),
            jax.ShapeDtypeStruct((BSC, MLP_D), jnp.float32),
            jax.ShapeDtypeStruct((BSC, MLP_D), jnp.float32),
        ],
        scratch_types=[
            pltpu.VMEM((BPW,), jnp.int32),
            pltpu.VMEM((BPW,), jnp.int32),
            pltpu.VMEM((RR, MF_D), jnp.float32),
            pltpu.VMEM((RR, MF_D), jnp.float32),
            pltpu.VMEM((RR, MLP_D), jnp.float32),
            pltpu.VMEM((RR, MLP_D), jnp.float32),
            pltpu.SemaphoreType.DMA,
        ],
    )
    def _sc_gather(u_hbm, i_hbm, mfu_hbm, mfi_hbm, mlpu_hbm, mlpi_hbm,
                   o_mfu, o_mfi, o_mlpu, o_mlpi,
                   uv, iv, bmfu, bmfi, bmlpu, bmlpi, sem):
        wid = lax.axis_index("s") * NC + lax.axis_index("c")
        base = wid * BPW
        pltpu.sync_copy(u_hbm.at[pl.ds(base, BPW)], uv)
        pltpu.sync_copy(i_hbm.at[pl.ds(base, BPW)], iv)

        def round_(t, _):
            r0 = t * RR

            def body(g, _):
                gr = r0 + g * 16
                uvec = uv[pl.ds(gr, 16)]
                ivec = iv[pl.ds(gr, 16)]
                for j in range(16):
                    r = g * 16 + j
                    a = uvec[j]
                    b = ivec[j]
                    pltpu.async_copy(mfu_hbm.at[pl.ds(a, 1)],
                                     bmfu.at[pl.ds(r, 1)], sem)
                    pltpu.async_copy(mfi_hbm.at[pl.ds(b, 1)],
                                     bmfi.at[pl.ds(r, 1)], sem)
                    pltpu.async_copy(mlpu_hbm.at[pl.ds(a, 1)],
                                     bmlpu.at[pl.ds(r, 1)], sem)
                    pltpu.async_copy(mlpi_hbm.at[pl.ds(b, 1)],
                                     bmlpi.at[pl.ds(r, 1)], sem)
                return _

            lax.fori_loop(0, RR // 16, body, 0)
            ob = base + r0
            pltpu.make_async_copy(o_mfu.at[pl.ds(ob, RR)], bmfu, sem).wait()
            pltpu.make_async_copy(o_mfi.at[pl.ds(ob, RR)], bmfi, sem).wait()
            pltpu.make_async_copy(o_mlpu.at[pl.ds(ob, RR)], bmlpu, sem).wait()
            pltpu.make_async_copy(o_mlpi.at[pl.ds(ob, RR)], bmlpi, sem).wait()
            pltpu.sync_copy(bmfu, o_mfu.at[pl.ds(ob, RR)])
            pltpu.sync_copy(bmfi, o_mfi.at[pl.ds(ob, RR)])
            pltpu.sync_copy(bmlpu, o_mlpu.at[pl.ds(ob, RR)])
            pltpu.sync_copy(bmlpi, o_mlpi.at[pl.ds(ob, RR)])
            return _

        lax.fori_loop(0, BPW // RR, round_, 0)

    return _sc_gather


def _tc_gather_body(u_smem, i_smem, mfu_hbm, mfi_hbm, mlpu_hbm, mlpi_hbm,
                    o_mfu, o_mfi, o_mlpu, o_mlpi,
                    bmfu, bmfi, bmlpu, bmlpi, sem):
    def body(r, _):
        a = u_smem[r]
        b = i_smem[r]
        pltpu.make_async_copy(mfu_hbm.at[pl.ds(a, 1)],
                              bmfu.at[pl.ds(r, 1)], sem).start()
        pltpu.make_async_copy(mfi_hbm.at[pl.ds(b, 1)],
                              bmfi.at[pl.ds(r, 1)], sem).start()
        pltpu.make_async_copy(mlpu_hbm.at[pl.ds(a, 1)],
                              bmlpu.at[pl.ds(r, 1)], sem).start()
        pltpu.make_async_copy(mlpi_hbm.at[pl.ds(b, 1)],
                              bmlpi.at[pl.ds(r, 1)], sem).start()
        return _

    lax.fori_loop(0, BTC, body, 0, unroll=8)
    pltpu.make_async_copy(mfu_hbm.at[pl.ds(0, BTC)], bmfu, sem).wait()
    pltpu.make_async_copy(mfi_hbm.at[pl.ds(0, BTC)], bmfi, sem).wait()
    pltpu.make_async_copy(mlpu_hbm.at[pl.ds(0, BTC)], bmlpu, sem).wait()
    pltpu.make_async_copy(mlpi_hbm.at[pl.ds(0, BTC)], bmlpi, sem).wait()
    o_mfu[...] = bmfu[...]
    o_mfi[...] = bmfi[...]
    o_mlpu[...] = bmlpu[...]
    o_mlpi[...] = bmlpi[...]


def _tc_gather(u2, i2, mf_user, mf_item, mlp_user, mlp_item):
    hbm = pl.BlockSpec(memory_space=pltpu.MemorySpace.HBM)
    smem = pl.BlockSpec(memory_space=pltpu.SMEM)
    return pl.pallas_call(
        _tc_gather_body,
        in_specs=[smem, smem, hbm, hbm, hbm, hbm],
        out_specs=[
            pl.BlockSpec((BTC, MF_D), lambda: (0, 0)),
            pl.BlockSpec((BTC, MF_D), lambda: (0, 0)),
            pl.BlockSpec((BTC, MLP_D), lambda: (0, 0)),
            pl.BlockSpec((BTC, MLP_D), lambda: (0, 0)),
        ],
        out_shape=[
            jax.ShapeDtypeStruct((BTC, MF_D), jnp.float32),
            jax.ShapeDtypeStruct((BTC, MF_D), jnp.float32),
            jax.ShapeDtypeStruct((BTC, MLP_D), jnp.float32),
            jax.ShapeDtypeStruct((BTC, MLP_D), jnp.float32),
        ],
        scratch_shapes=[
            pltpu.VMEM((BTC, MF_D), jnp.float32),
            pltpu.VMEM((BTC, MF_D), jnp.float32),
            pltpu.VMEM((BTC, MLP_D), jnp.float32),
            pltpu.VMEM((BTC, MLP_D), jnp.float32),
            pltpu.SemaphoreType.DMA,
        ],
    )(u2, i2, mf_user, mf_item, mlp_user, mlp_item)


BM = 2048


def _tc_mlp_body(mfu, mfi, mlpu, mlpi, W1a, W1b, b1, W2, b2, W3, b3,
                 Wl, bl, w2la, w2lb, b2l, out):
    f32 = jnp.float32
    x = (jnp.dot(mlpu[...], W1a[...], preferred_element_type=f32)
         + jnp.dot(mlpi[...], W1b[...], preferred_element_type=f32)
         + b1[...])
    x = jnp.maximum(x, 0.0)
    x = jnp.dot(x, W2[...], preferred_element_type=f32) + b2[...]
    x = jnp.maximum(x, 0.0)
    x = jnp.dot(x, W3[...], preferred_element_type=f32) + b3[...]
    x = jnp.maximum(x, 0.0)
    mlp_vec = jnp.dot(x, Wl[...], preferred_element_type=f32) + bl[...]
    mf_vec = mfu[...] * mfi[...]
    logit = (jnp.dot(mf_vec, w2la[...], preferred_element_type=f32)
             + jnp.dot(mlp_vec, w2lb[...], preferred_element_type=f32)
             + b2l[...])
    out[...] = jax.nn.sigmoid(logit)


def _tc_mlp(mfu, mfi, mlpu, mlpi, W1a, W1b, b1, W2, b2, W3, b3,
            Wl, bl, w2la, w2lb, b2l):
    def row_block(d):
        return pl.BlockSpec((BM, d), lambda m: (m, 0))

    def full(a):
        return pl.BlockSpec(a.shape, lambda m: (0,) * a.ndim)

    return pl.pallas_call(
        _tc_mlp_body,
        grid=(B // BM,),
        in_specs=[
            row_block(MF_D), row_block(MF_D), row_block(MLP_D),
            row_block(MLP_D),
            full(W1a), full(W1b), full(b1), full(W2), full(b2),
            full(W3), full(b3), full(Wl), full(bl),
            full(w2la), full(w2lb), full(b2l),
        ],
        out_specs=pl.BlockSpec((BM, 1), lambda m: (m, 0)),
        out_shape=jax.ShapeDtypeStruct((B, 1), jnp.float32),
    )(mfu, mfi, mlpu, mlpi, W1a, W1b, b1, W2, b2, W3, b3,
      Wl, bl, w2la, w2lb, b2l)


def kernel(inputs, mf_user, mf_item, mlp_user, mlp_item,
           W1, b1, W2, b2, W3, b3, Wl, bl, W2l, b2l):
    u = inputs[:, 0]
    i = inputs[:, 1]
    mfu1, mfi1, mlpu1, mlpi1 = _make_sc_gather()(
        u[:BSC], i[:BSC], mf_user, mf_item, mlp_user, mlp_item)
    mfu2, mfi2, mlpu2, mlpi2 = _tc_gather(
        u[BSC:], i[BSC:], mf_user, mf_item, mlp_user, mlp_item)
    mfu = jnp.concatenate([mfu1, mfu2], axis=0)
    mfi = jnp.concatenate([mfi1, mfi2], axis=0)
    mlpu = jnp.concatenate([mlpu1, mlpu2], axis=0)
    mlpi = jnp.concatenate([mlpi1, mlpi2], axis=0)
    return _tc_mlp(
        mfu, mfi, mlpu, mlpi,
        W1[:MLP_D], W1[MLP_D:], b1.reshape(1, -1),
        W2, b2.reshape(1, -1), W3, b3.reshape(1, -1),
        Wl, bl.reshape(1, -1),
        W2l[:MF_D], W2l[MF_D:], b2l.reshape(1, 1),
    )


# speedup vs baseline: 1.0584x; 1.0005x over previous
import functools

import jax
import jax.numpy as jnp
from jax import lax
from jax.experimental import pallas as pl
from jax.experimental.pallas import tpu as pltpu
from jax.experimental.pallas import tpu_sc as plsc

B = 16384
MF_D = 16
MLP_D = 10
NC = 2
NS = 16
NW = NC * NS
BSC = 12288
BTC = B - BSC
BPW = BSC // NW
RR = 128


@functools.cache
def _make_sc_gather():
    mesh = plsc.VectorSubcoreMesh(core_axis_name="c", subcore_axis_name="s")

    @functools.partial(
        pl.kernel,
        mesh=mesh,
        compiler_params=pltpu.CompilerParams(use_tc_tiling_on_sc=True),
        out_type=[
            jax.ShapeDtypeStruct((BSC, MF_D), jnp.float32),
            jax.ShapeDtypeStruct((BSC, MF_D), jnp.float32),
            jax.ShapeDtypeStruct((BSC, MLP_D), jnp.float32),
            jax.ShapeDtypeStruct((BSC, MLP_D), jnp.float32),
        ],
        scratch_types=[
            pltpu.VMEM((BPW,), jnp.int32),
            pltpu.VMEM((BPW,), jnp.int32),
            pltpu.VMEM((RR, MF_D), jnp.float32),
            pltpu.VMEM((RR, MF_D), jnp.float32),
            pltpu.VMEM((RR, MLP_D), jnp.float32),
            pltpu.VMEM((RR, MLP_D), jnp.float32),
            pltpu.SemaphoreType.DMA,
        ],
    )
    def _sc_gather(u_hbm, i_hbm, mfu_hbm, mfi_hbm, mlpu_hbm, mlpi_hbm,
                   o_mfu, o_mfi, o_mlpu, o_mlpi,
                   uv, iv, bmfu, bmfi, bmlpu, bmlpi, sem):
        wid = lax.axis_index("s") * NC + lax.axis_index("c")
        base = wid * BPW
        pltpu.sync_copy(u_hbm.at[pl.ds(base, BPW)], uv)
        pltpu.sync_copy(i_hbm.at[pl.ds(base, BPW)], iv)

        def round_(t, _):
            r0 = t * RR

            def body(g, _):
                gr = r0 + g * 16
                uvec = uv[pl.ds(gr, 16)]
                ivec = iv[pl.ds(gr, 16)]
                for j in range(16):
                    r = g * 16 + j
                    a = uvec[j]
                    b = ivec[j]
                    pltpu.async_copy(mfu_hbm.at[pl.ds(a, 1)],
                                     bmfu.at[pl.ds(r, 1)], sem)
                    pltpu.async_copy(mfi_hbm.at[pl.ds(b, 1)],
                                     bmfi.at[pl.ds(r, 1)], sem)
                    pltpu.async_copy(mlpu_hbm.at[pl.ds(a, 1)],
                                     bmlpu.at[pl.ds(r, 1)], sem)
                    pltpu.async_copy(mlpi_hbm.at[pl.ds(b, 1)],
                                     bmlpi.at[pl.ds(r, 1)], sem)
                return _

            lax.fori_loop(0, RR // 16, body, 0)
            ob = base + r0
            pltpu.make_async_copy(o_mfu.at[pl.ds(ob, RR)], bmfu, sem).wait()
            pltpu.make_async_copy(o_mfi.at[pl.ds(ob, RR)], bmfi, sem).wait()
            pltpu.make_async_copy(o_mlpu.at[pl.ds(ob, RR)], bmlpu, sem).wait()
            pltpu.make_async_copy(o_mlpi.at[pl.ds(ob, RR)], bmlpi, sem).wait()
            pltpu.sync_copy(bmfu, o_mfu.at[pl.ds(ob, RR)])
            pltpu.sync_copy(bmfi, o_mfi.at[pl.ds(ob, RR)])
            pltpu.sync_copy(bmlpu, o_mlpu.at[pl.ds(ob, RR)])
            pltpu.sync_copy(bmlpi, o_mlpi.at[pl.ds(ob, RR)])
            return _

        lax.fori_loop(0, BPW // RR, round_, 0)

    return _sc_gather


def _tc_gather_body(u_smem, i_smem, mfu_hbm, mfi_hbm, mlpu_hbm, mlpi_hbm,
                    o_mfu, o_mfi, o_mlpu, o_mlpi,
                    bmfu, bmfi, bmlpu, bmlpi, sem):
    def body(r, _):
        a = u_smem[r]
        b = i_smem[r]
        pltpu.make_async_copy(mfu_hbm.at[pl.ds(a, 1)],
                              bmfu.at[pl.ds(r, 1)], sem).start()
        pltpu.make_async_copy(mfi_hbm.at[pl.ds(b, 1)],
                              bmfi.at[pl.ds(r, 1)], sem).start()
        pltpu.make_async_copy(mlpu_hbm.at[pl.ds(a, 1)],
                              bmlpu.at[pl.ds(r, 1)], sem).start()
        pltpu.make_async_copy(mlpi_hbm.at[pl.ds(b, 1)],
                              bmlpi.at[pl.ds(r, 1)], sem).start()
        return _

    lax.fori_loop(0, BTC, body, 0, unroll=8)
    pltpu.make_async_copy(mfu_hbm.at[pl.ds(0, BTC)], bmfu, sem).wait()
    pltpu.make_async_copy(mfi_hbm.at[pl.ds(0, BTC)], bmfi, sem).wait()
    pltpu.make_async_copy(mlpu_hbm.at[pl.ds(0, BTC)], bmlpu, sem).wait()
    pltpu.make_async_copy(mlpi_hbm.at[pl.ds(0, BTC)], bmlpi, sem).wait()
    o_mfu[...] = bmfu[...]
    o_mfi[...] = bmfi[...]
    o_mlpu[...] = bmlpu[...]
    o_mlpi[...] = bmlpi[...]


def _tc_gather(u2, i2, mf_user, mf_item, mlp_user, mlp_item):
    hbm = pl.BlockSpec(memory_space=pltpu.MemorySpace.HBM)
    smem = pl.BlockSpec(memory_space=pltpu.SMEM)
    return pl.pallas_call(
        _tc_gather_body,
        in_specs=[smem, smem, hbm, hbm, hbm, hbm],
        out_specs=[
            pl.BlockSpec((BTC, MF_D), lambda: (0, 0)),
            pl.BlockSpec((BTC, MF_D), lambda: (0, 0)),
            pl.BlockSpec((BTC, MLP_D), lambda: (0, 0)),
            pl.BlockSpec((BTC, MLP_D), lambda: (0, 0)),
        ],
        out_shape=[
            jax.ShapeDtypeStruct((BTC, MF_D), jnp.float32),
            jax.ShapeDtypeStruct((BTC, MF_D), jnp.float32),
            jax.ShapeDtypeStruct((BTC, MLP_D), jnp.float32),
            jax.ShapeDtypeStruct((BTC, MLP_D), jnp.float32),
        ],
        scratch_shapes=[
            pltpu.VMEM((BTC, MF_D), jnp.float32),
            pltpu.VMEM((BTC, MF_D), jnp.float32),
            pltpu.VMEM((BTC, MLP_D), jnp.float32),
            pltpu.VMEM((BTC, MLP_D), jnp.float32),
            pltpu.SemaphoreType.DMA,
        ],
    )(u2, i2, mf_user, mf_item, mlp_user, mlp_item)


BM = 2048


def _tc_mlp_body(mfu, mfi, mlpu, mlpi, W1a, W1b, b1, W2, b2, W3, b3,
                 Wl, bl, w2la, w2lb, b2l, out):
    f32 = jnp.float32
    x = (jnp.dot(mlpu[...], W1a[...], preferred_element_type=f32)
         + jnp.dot(mlpi[...], W1b[...], preferred_element_type=f32)
         + b1[...])
    x = jnp.maximum(x, 0.0)
    x = jnp.dot(x, W2[...], preferred_element_type=f32) + b2[...]
    x = jnp.maximum(x, 0.0)
    x = jnp.dot(x, W3[...], preferred_element_type=f32) + b3[...]
    x = jnp.maximum(x, 0.0)
    mlp_vec = jnp.dot(x, Wl[...], preferred_element_type=f32) + bl[...]
    mf_vec = mfu[...] * mfi[...]
    logit = (jnp.dot(mf_vec, w2la[...], preferred_element_type=f32)
             + jnp.dot(mlp_vec, w2lb[...], preferred_element_type=f32)
             + b2l[...])
    out[...] = jax.nn.sigmoid(logit)


def _tc_mlp(mfu, mfi, mlpu, mlpi, W1a, W1b, b1, W2, b2, W3, b3,
            Wl, bl, w2la, w2lb, b2l):
    def row_block(d):
        return pl.BlockSpec((BM, d), lambda m: (m, 0))

    def full(a):
        return pl.BlockSpec(a.shape, lambda m: (0,) * a.ndim)

    return pl.pallas_call(
        _tc_mlp_body,
        grid=(B // BM,),
        in_specs=[
            row_block(MF_D), row_block(MF_D), row_block(MLP_D),
            row_block(MLP_D),
            full(W1a), full(W1b), full(b1), full(W2), full(b2),
            full(W3), full(b3), full(Wl), full(bl),
            full(w2la), full(w2lb), full(b2l),
        ],
        out_specs=pl.BlockSpec((BM, 1), lambda m: (m, 0)),
        out_shape=jax.ShapeDtypeStruct((B, 1), jnp.float32),
    )(mfu, mfi, mlpu, mlpi, W1a, W1b, b1, W2, b2, W3, b3,
      Wl, bl, w2la, w2lb, b2l)


def kernel(inputs, mf_user, mf_item, mlp_user, mlp_item,
           W1, b1, W2, b2, W3, b3, Wl, bl, W2l, b2l):
    u = inputs[:, 0]
    i = inputs[:, 1]
    mfu2, mfi2, mlpu2, mlpi2 = _tc_gather(
        u[BSC:], i[BSC:], mf_user, mf_item, mlp_user, mlp_item)
    mfu1, mfi1, mlpu1, mlpi1 = _make_sc_gather()(
        u[:BSC], i[:BSC], mf_user, mf_item, mlp_user, mlp_item)
    mfu = jnp.concatenate([mfu1, mfu2], axis=0)
    mfi = jnp.concatenate([mfi1, mfi2], axis=0)
    mlpu = jnp.concatenate([mlpu1, mlpu2], axis=0)
    mlpi = jnp.concatenate([mlpi1, mlpi2], axis=0)
    return _tc_mlp(
        mfu, mfi, mlpu, mlpi,
        W1[:MLP_D], W1[MLP_D:], b1.reshape(1, -1),
        W2, b2.reshape(1, -1), W3, b3.reshape(1, -1),
        Wl, bl.reshape(1, -1),
        W2l[:MF_D], W2l[MF_D:], b2l.reshape(1, 1),
    )


# speedup vs baseline: 1.1459x; 1.0827x over previous
import functools

import jax
import jax.numpy as jnp
from jax import lax
from jax.experimental import pallas as pl
from jax.experimental.pallas import tpu as pltpu
from jax.experimental.pallas import tpu_sc as plsc

B = 16384
MF_D = 16
MLP_D = 10
NC = 2
NS = 16
NW = NC * NS
BPW = B // NW
RR = 128


@functools.cache
def _make_sc_gather():
    mesh = plsc.VectorSubcoreMesh(core_axis_name="c", subcore_axis_name="s")

    @functools.partial(
        pl.kernel,
        mesh=mesh,
        compiler_params=pltpu.CompilerParams(use_tc_tiling_on_sc=True),
        out_type=[
            jax.ShapeDtypeStruct((B, MF_D), jnp.float32),
            jax.ShapeDtypeStruct((B, MF_D), jnp.float32),
            jax.ShapeDtypeStruct((B, MLP_D), jnp.float32),
            jax.ShapeDtypeStruct((B, MLP_D), jnp.float32),
        ],
        scratch_types=[
            pltpu.VMEM((BPW,), jnp.int32),
            pltpu.VMEM((BPW,), jnp.int32),
            pltpu.VMEM((RR, MF_D), jnp.float32),
            pltpu.VMEM((RR, MF_D), jnp.float32),
            pltpu.VMEM((RR, MLP_D), jnp.float32),
            pltpu.VMEM((RR, MLP_D), jnp.float32),
            pltpu.SemaphoreType.DMA,
        ],
    )
    def _sc_gather(u_hbm, i_hbm, mfu_hbm, mfi_hbm, mlpu_hbm, mlpi_hbm,
                   o_mfu, o_mfi, o_mlpu, o_mlpi,
                   uv, iv, bmfu, bmfi, bmlpu, bmlpi, sem):
        wid = lax.axis_index("s") * NC + lax.axis_index("c")
        base = wid * BPW
        pltpu.sync_copy(u_hbm.at[pl.ds(base, BPW)], uv)
        pltpu.sync_copy(i_hbm.at[pl.ds(base, BPW)], iv)

        def round_(t, _):
            r0 = t * RR

            def body(g, _):
                gr = r0 + g * 16
                uvec = uv[pl.ds(gr, 16)]
                ivec = iv[pl.ds(gr, 16)]
                for j in range(16):
                    r = g * 16 + j
                    a = uvec[j]
                    b = ivec[j]
                    pltpu.async_copy(mfu_hbm.at[pl.ds(a, 1)],
                                     bmfu.at[pl.ds(r, 1)], sem)
                    pltpu.async_copy(mfi_hbm.at[pl.ds(b, 1)],
                                     bmfi.at[pl.ds(r, 1)], sem)
                    pltpu.async_copy(mlpu_hbm.at[pl.ds(a, 1)],
                                     bmlpu.at[pl.ds(r, 1)], sem)
                    pltpu.async_copy(mlpi_hbm.at[pl.ds(b, 1)],
                                     bmlpi.at[pl.ds(r, 1)], sem)
                return _

            lax.fori_loop(0, RR // 16, body, 0)
            ob = base + r0
            pltpu.make_async_copy(o_mfu.at[pl.ds(ob, RR)], bmfu, sem).wait()
            pltpu.make_async_copy(o_mfi.at[pl.ds(ob, RR)], bmfi, sem).wait()
            pltpu.make_async_copy(o_mlpu.at[pl.ds(ob, RR)], bmlpu, sem).wait()
            pltpu.make_async_copy(o_mlpi.at[pl.ds(ob, RR)], bmlpi, sem).wait()
            pltpu.sync_copy(bmfu, o_mfu.at[pl.ds(ob, RR)])
            pltpu.sync_copy(bmfi, o_mfi.at[pl.ds(ob, RR)])
            pltpu.sync_copy(bmlpu, o_mlpu.at[pl.ds(ob, RR)])
            pltpu.sync_copy(bmlpi, o_mlpi.at[pl.ds(ob, RR)])
            return _

        lax.fori_loop(0, BPW // RR, round_, 0)

    return _sc_gather


BM = 2048


def _tc_mlp_body(mfu, mfi, mlpu, mlpi, W1a, W1b, b1, W2, b2, W3, b3,
                 Wl, bl, w2la, w2lb, b2l, out):
    f32 = jnp.float32
    x = (jnp.dot(mlpu[...], W1a[...], preferred_element_type=f32)
         + jnp.dot(mlpi[...], W1b[...], preferred_element_type=f32)
         + b1[...])
    x = jnp.maximum(x, 0.0)
    x = jnp.dot(x, W2[...], preferred_element_type=f32) + b2[...]
    x = jnp.maximum(x, 0.0)
    x = jnp.dot(x, W3[...], preferred_element_type=f32) + b3[...]
    x = jnp.maximum(x, 0.0)
    mlp_vec = jnp.dot(x, Wl[...], preferred_element_type=f32) + bl[...]
    mf_vec = mfu[...] * mfi[...]
    logit = (jnp.dot(mf_vec, w2la[...], preferred_element_type=f32)
             + jnp.dot(mlp_vec, w2lb[...], preferred_element_type=f32)
             + b2l[...])
    out[...] = jax.nn.sigmoid(logit)


def _tc_mlp(mfu, mfi, mlpu, mlpi, W1a, W1b, b1, W2, b2, W3, b3,
            Wl, bl, w2la, w2lb, b2l):
    def row_block(d):
        return pl.BlockSpec((BM, d), lambda m: (m, 0))

    def full(a):
        return pl.BlockSpec(a.shape, lambda m: (0,) * a.ndim)

    return pl.pallas_call(
        _tc_mlp_body,
        grid=(B // BM,),
        in_specs=[
            row_block(MF_D), row_block(MF_D), row_block(MLP_D),
            row_block(MLP_D),
            full(W1a), full(W1b), full(b1), full(W2), full(b2),
            full(W3), full(b3), full(Wl), full(bl),
            full(w2la), full(w2lb), full(b2l),
        ],
        out_specs=pl.BlockSpec((BM, 1), lambda m: (m, 0)),
        out_shape=jax.ShapeDtypeStruct((B, 1), jnp.float32),
    )(mfu, mfi, mlpu, mlpi, W1a, W1b, b1, W2, b2, W3, b3,
      Wl, bl, w2la, w2lb, b2l)


def kernel(inputs, mf_user, mf_item, mlp_user, mlp_item,
           W1, b1, W2, b2, W3, b3, Wl, bl, W2l, b2l):
    u = inputs[:, 0]
    i = inputs[:, 1]
    mfu, mfi, mlpu, mlpi = _make_sc_gather()(
        u, i, mf_user, mf_item, mlp_user, mlp_item)
    return _tc_mlp(
        mfu, mfi, mlpu, mlpi,
        W1[:MLP_D], W1[MLP_D:], b1.reshape(1, -1),
        W2, b2.reshape(1, -1), W3, b3.reshape(1, -1),
        Wl, bl.reshape(1, -1),
        W2l[:MF_D], W2l[MF_D:], b2l.reshape(1, 1),
    )
